# Initial kernel scaffold; baseline (speedup 1.0000x reference)
import jax
import jax.numpy as jnp
from jax.experimental import pallas as pl


def kernel(state, edge_index, batch_size, rej_rate, theta_value, W1, b1, W2, b2, W3, b3, rW1, rb1, rW2, rb2, rW3, rb3, fcW, fcb, g1, be1, g2, be2, g3, be3, go, bo):
    raise NotImplementedError("write your pallas kernel here")


# speedup vs baseline: 16.6202x; 16.6202x over previous
import jax
import jax.numpy as jnp
from jax import lax
from jax.experimental import pallas as pl
from jax.experimental.pallas import tpu as pltpu

_N = 64
_F = 16
_BG = 8


def _layer(Ab, xin, W, b, rW, rb, g, be):
    mix = jnp.dot(Ab, xin, preferred_element_type=jnp.float32)
    h = (jnp.dot(mix, W, preferred_element_type=jnp.float32) + b
         + jnp.dot(xin, rW, preferred_element_type=jnp.float32) + rb)
    mu = jnp.mean(h, axis=1, keepdims=True)
    var = jnp.mean((h - mu) ** 2, axis=1, keepdims=True)
    hn = (h - mu) * lax.rsqrt(var + 1e-5) * g + be
    return jnp.where(hn > 0, hn, 0.01 * hn)


def _body(x_ref, e_ref, et_ref, W1r, b1r, W2r, b2r, W3r, b3r,
          rW1r, rb1r, rW2r, rb2r, rW3r, rb3r, fcWr, fcbr,
          g1r, be1r, g2r, be2r, g3r, be3r, gor, bor, unit_ref,
          out_ref, Abig_ref):
    R = _BG * _N
    i = pl.program_id(0)

    @pl.when(i == 0)
    def _build():
        E = e_ref.shape[1]
        f32 = jnp.float32
        d_row = e_ref[1:2, :]
        s_col = et_ref[:, 0:1]
        d_col = et_ref[:, 1:2]
        ii = lax.broadcasted_iota(jnp.int32, (_N, E), 0)
        D = (d_row == ii).astype(f32)
        jj = lax.broadcasted_iota(jnp.int32, (E, _N), 1)
        S_T = (s_col == jj).astype(f32)
        D_T = (d_col == jj).astype(f32)
        r64 = lax.broadcasted_iota(jnp.int32, (_N, _N), 0)
        c64 = lax.broadcasted_iota(jnp.int32, (_N, _N), 1)
        eye = (r64 == c64).astype(f32)
        C = jnp.dot(D, S_T, preferred_element_type=f32) + eye
        deg_c = jnp.sum(D, axis=1, keepdims=True) + 1.0
        deg_r = jnp.sum(D_T, axis=0, keepdims=True) + 1.0
        A = C * lax.rsqrt(deg_c) * lax.rsqrt(deg_r)
        p0 = lax.broadcasted_iota(jnp.int32, (R, _N), 0)
        i1 = lax.broadcasted_iota(jnp.int32, (R, _N), 1)
        E2 = ((p0 & (_N - 1)) == i1).astype(f32)
        i2 = lax.broadcasted_iota(jnp.int32, (_N, R), 0)
        q1 = lax.broadcasted_iota(jnp.int32, (_N, R), 1)
        E2T = (i2 == (q1 & (_N - 1))).astype(f32)
        pg = lax.broadcasted_iota(jnp.int32, (R, R), 0) >> 6
        qg = lax.broadcasted_iota(jnp.int32, (R, R), 1) >> 6
        mask = (pg == qg).astype(f32)
        Abig_ref[...] = jnp.dot(jnp.dot(E2, A, preferred_element_type=f32),
                                E2T, preferred_element_type=f32) * mask

    Ab = Abig_ref[...]
    x = x_ref[...]
    x1 = _layer(Ab, x, W1r[...], b1r[...], rW1r[...], rb1r[...],
                g1r[...], be1r[...])
    x2 = _layer(Ab, x1, W2r[...], b2r[...], rW2r[...], rb2r[...],
                g2r[...], be2r[...])
    x3 = _layer(Ab, x2, W3r[...], b3r[...], rW3r[...], rb3r[...],
                g3r[...], be3r[...])
    h4 = (jnp.dot(x3, fcWr[0:256, :], preferred_element_type=jnp.float32)
          + jnp.dot(x, fcWr[256:272, :], preferred_element_type=jnp.float32)
          + fcbr[...])
    mu = jnp.mean(h4, axis=1, keepdims=True)
    var = jnp.mean((h4 - mu) ** 2, axis=1, keepdims=True)
    hn = (h4 - mu) * lax.rsqrt(var + 1e-5) * gor[...] + bor[...]
    y = jnp.tanh(hn) * unit_ref[...]
    out_ref[...] = y


def kernel(state, edge_index, batch_size, rej_rate, theta_value,
           W1, b1, W2, b2, W3, b3, rW1, rb1, rW2, rb2, rW3, rb3, fcW, fcb,
           g1, be1, g2, be2, g3, be3, go, bo):
    B = state.shape[0] // _N
    E = edge_index.shape[1]
    R = _BG * _N
    grid = B // _BG
    f32 = jnp.float32
    e = edge_index.astype(jnp.int32)
    et = e.T
    unit = (jnp.asarray(batch_size).astype(f32) / jnp.asarray(B, f32)
            ).reshape(1, 1)
    row = lambda v: v.reshape(1, -1).astype(f32)
    full = lambda a: pl.BlockSpec(a.shape, lambda i: (0,) * a.ndim)
    args = (e, et, W1, row(b1), W2, row(b2), W3, row(b3),
            rW1, row(rb1), rW2, row(rb2), rW3, row(rb3), fcW, row(fcb),
            row(g1), row(be1), row(g2), row(be2), row(g3), row(be3),
            row(go), row(bo), unit)
    out = pl.pallas_call(
        _body,
        grid=(grid,),
        in_specs=[pl.BlockSpec((R, _F), lambda i: (i, 0))]
                 + [full(a) for a in args],
        out_specs=pl.BlockSpec((R, 4), lambda i: (i, 0)),
        out_shape=jax.ShapeDtypeStruct((B * _N, 4), f32),
        scratch_shapes=[pltpu.VMEM((R, R), f32)],
    )(state.astype(f32), *args)
    return out.reshape(B, _N * 4)
